# Initial kernel scaffold; baseline (speedup 1.0000x reference)
import jax
import jax.numpy as jnp
from jax.experimental import pallas as pl


def kernel(input_ids, token_type_ids, word_table, word_scale, pos_table, pos_scale, type_table, type_scale, ln_weight, ln_bias):
    raise NotImplementedError("write your pallas kernel here")


# speedup vs baseline: 2.1368x; 2.1368x over previous
import functools

import jax
import jax.numpy as jnp
from jax import lax
from jax.experimental import pallas as pl
from jax.experimental.pallas import tpu as pltpu
from jax.experimental.pallas import tpu_sc as plsc

_VOCAB = 30522
_HIDDEN = 768
_SEQ = 128
_BATCH = 1024
_EPS = 1e-12
_NR_ITERS = 8

_ROW_W = _HIDDEN // 4
_TOK = _BATCH * _SEQ
_NW = 32
_TPW = _TOK // _NW
_CH = 128
_NCH = _TPW // _CH

_BB = 4
_TPB = _BB * _SEQ
_NB = _BATCH // _BB


def _sc_gather(ids3, table_i32):
    info = plsc.get_sparse_core_info()
    nc = info.num_cores

    mesh = plsc.VectorSubcoreMesh(core_axis_name="c", subcore_axis_name="s")

    @functools.partial(
        pl.kernel,
        mesh=mesh,
        compiler_params=pltpu.CompilerParams(use_tc_tiling_on_sc=False),
        out_type=jax.ShapeDtypeStruct((_TOK, _ROW_W), jnp.int32),
        scratch_types=[
            pltpu.VMEM((_NCH, _CH), jnp.int32),
            pltpu.VMEM((2, _CH, _ROW_W), jnp.int32),
            pltpu.SemaphoreType.DMA,
            pltpu.SemaphoreType.DMA,
        ],
    )
    def gk(ids_hbm, tab_hbm, out_hbm, idx_v, rows_v, gsem, ssem):
        wid = lax.axis_index("s") * nc + lax.axis_index("c")
        base = wid * _TPW
        pltpu.sync_copy(ids_hbm.at[wid], idx_v)

        pltpu.async_copy(tab_hbm.at[idx_v.at[0]], rows_v.at[0], gsem)

        def body(c, carry):
            buf = lax.rem(c, 2)

            @pl.when(c + 1 < _NCH)
            def _prefetch():
                pltpu.async_copy(
                    tab_hbm.at[idx_v.at[c + 1]], rows_v.at[1 - buf], gsem
                )

            pltpu.make_async_copy(
                tab_hbm.at[idx_v.at[c]], rows_v.at[buf], gsem
            ).wait()
            pltpu.async_copy(
                rows_v.at[buf], out_hbm.at[pl.ds(base + c * _CH, _CH)], ssem
            ).wait()
            return carry

        lax.fori_loop(0, _NCH, body, 0)

    return gk(ids3, table_i32)


def _ln_body(ws_ref, tt_ref, g_ref, ptc_ref, dt_ref, lnw_ref, lnb_ref, out_ref):
    ws = ws_ref[0]
    w8 = g_ref[0]
    ptc = jnp.broadcast_to(
        ptc_ref[...][None], (_BB, _SEQ, _HIDDEN)
    ).reshape(_TPB, _HIDDEN)
    ttf = tt_ref[0].astype(jnp.float32)
    e = w8.astype(jnp.float32) * ws + ptc + ttf * dt_ref[...]
    mean = jnp.mean(e, axis=1, keepdims=True)
    d = e - mean
    var = jnp.mean(d * d, axis=1, keepdims=True)
    x = jnp.where(var > 1.0, var * 0.5, jnp.ones_like(var))
    for _ in range(_NR_ITERS):
        x = 0.5 * (x + var / (x + 1e-9))
    r = 1.0 / (x + _EPS)
    out_ref[0] = (d * r) * lnw_ref[...] + lnb_ref[...]


def kernel(input_ids, token_type_ids, word_table, word_scale, pos_table,
           pos_scale, type_table, type_scale, ln_weight, ln_bias):
    ids = input_ids.astype(jnp.int32).reshape(_NW, _NCH, _CH)
    table_i32 = lax.bitcast_convert_type(
        word_table.reshape(_VOCAB, _ROW_W, 4), jnp.int32
    )

    gathered = _sc_gather(ids, table_i32)
    g8 = lax.bitcast_convert_type(gathered, jnp.int8).reshape(_NB, _TPB, _HIDDEN)

    posf = pos_table[:_SEQ].astype(jnp.float32) * pos_scale
    t0 = type_table[0].astype(jnp.float32) * type_scale
    t1 = type_table[1].astype(jnp.float32) * type_scale
    ptc = posf + t0[None, :]
    dt = (t1 - t0)[None, :]
    tt3 = token_type_ids.astype(jnp.int32).reshape(_NB, _TPB, 1)
    ws1 = word_scale.reshape(1)

    out = pl.pallas_call(
        _ln_body,
        grid=(_NB,),
        in_specs=[
            pl.BlockSpec(memory_space=pltpu.SMEM),
            pl.BlockSpec((1, _TPB, 1), lambda i: (i, 0, 0)),
            pl.BlockSpec((1, _TPB, _HIDDEN), lambda i: (i, 0, 0)),
            pl.BlockSpec((_SEQ, _HIDDEN), lambda i: (0, 0)),
            pl.BlockSpec((1, _HIDDEN), lambda i: (0, 0)),
            pl.BlockSpec((1, _HIDDEN), lambda i: (0, 0)),
            pl.BlockSpec((1, _HIDDEN), lambda i: (0, 0)),
        ],
        out_specs=pl.BlockSpec((1, _TPB, _HIDDEN), lambda i: (i, 0, 0)),
        out_shape=jax.ShapeDtypeStruct((_NB, _TPB, _HIDDEN), jnp.float32),
    )(ws1, tt3, g8, ptc, dt, ln_weight.reshape(1, _HIDDEN),
      ln_bias.reshape(1, _HIDDEN))

    return out.reshape(_BATCH, _SEQ, _HIDDEN)


# speedup vs baseline: 4.8867x; 2.2869x over previous
import functools

import jax
import jax.numpy as jnp
from jax import lax
from jax.experimental import pallas as pl
from jax.experimental.pallas import tpu as pltpu
from jax.experimental.pallas import tpu_sc as plsc

_VOCAB = 30522
_HIDDEN = 768
_SEQ = 128
_BATCH = 1024
_EPS = 1e-12
_NR_ITERS = 8

_QH = _HIDDEN // 4
_RW = 256
_TOK = _BATCH * _SEQ
_NW = 32
_TPW = _TOK // _NW
_CH = 128
_NCH = _TPW // _CH

_BB = 8
_NB = _BATCH // _BB


def _sc_gather(ids3, table_i32):
    info = plsc.get_sparse_core_info()
    nc = info.num_cores

    mesh = plsc.VectorSubcoreMesh(core_axis_name="c", subcore_axis_name="s")

    @functools.partial(
        pl.kernel,
        mesh=mesh,
        out_type=jax.ShapeDtypeStruct((_TOK, _RW), jnp.int32),
        scratch_types=[
            pltpu.VMEM((_NCH, _CH), jnp.int32),
            pltpu.VMEM((2, _CH, _RW), jnp.int32),
            pltpu.SemaphoreType.DMA,
            pltpu.SemaphoreType.DMA,
        ],
    )
    def gk(ids_hbm, tab_hbm, out_hbm, idx_v, rows_v, gsem, ssem):
        wid = lax.axis_index("s") * nc + lax.axis_index("c")
        base = wid * _TPW
        pltpu.sync_copy(ids_hbm.at[wid], idx_v)

        pltpu.async_copy(tab_hbm.at[idx_v.at[0]], rows_v.at[0], gsem)

        def body(c, carry):
            buf = lax.rem(c, 2)

            @pl.when(c + 1 < _NCH)
            def _prefetch():
                pltpu.async_copy(
                    tab_hbm.at[idx_v.at[c + 1]], rows_v.at[1 - buf], gsem
                )

            pltpu.make_async_copy(
                tab_hbm.at[idx_v.at[c]], rows_v.at[buf], gsem
            ).wait()
            pltpu.async_copy(
                rows_v.at[buf], out_hbm.at[pl.ds(base + c * _CH, _CH)], ssem
            ).wait()
            return carry

        lax.fori_loop(0, _NCH, body, 0)

    return gk(ids3, table_i32)


def _ln_body(ws_ref, g_ref, ptc_ref, dt_ref, lnw_ref, lnb_ref, out_ref):
    w32 = g_ref[...]
    ws = ws_ref[0]
    q = w32[:, :, :_QH]
    ttf = w32[:, :, _QH:_QH + 1].astype(jnp.float32)
    pieces = []
    for m in range(4):
        b = lax.shift_right_arithmetic(
            lax.shift_left(q, 24 - 8 * m), 24
        ).astype(jnp.float32)
        ptc_m = ptc_ref[:, m * _QH:(m + 1) * _QH]
        dt_m = dt_ref[:, m * _QH:(m + 1) * _QH]
        pieces.append(b * ws + ptc_m[None] + ttf * dt_m[None])
    e = jnp.concatenate(pieces, axis=2)
    mean = jnp.mean(e, axis=2, keepdims=True)
    d = e - mean
    var = jnp.mean(d * d, axis=2, keepdims=True)
    x = jnp.where(var > 1.0, var * 0.5, jnp.ones_like(var))
    for _ in range(_NR_ITERS):
        x = 0.5 * (x + var / (x + 1e-9))
    r = 1.0 / (x + _EPS)
    out_ref[...] = (d * r) * lnw_ref[...][None] + lnb_ref[...][None]


def kernel(input_ids, token_type_ids, word_table, word_scale, pos_table,
           pos_scale, type_table, type_scale, ln_weight, ln_bias):
    ids_eff = (input_ids.astype(jnp.int32) * 2
               + token_type_ids.astype(jnp.int32)).reshape(_NW, _NCH, _CH)

    wt_t = word_table.reshape(_VOCAB, 4, _QH).transpose(0, 2, 1)
    wt_i32 = lax.bitcast_convert_type(wt_t, jnp.int32)
    pad = jnp.zeros((_VOCAB, _RW - _QH), jnp.int32)
    tbit = jnp.concatenate(
        [jnp.zeros((_VOCAB, 1), jnp.int32), jnp.ones((_VOCAB, 1), jnp.int32)],
        axis=1,
    )
    ext = jnp.concatenate(
        [
            jnp.broadcast_to(wt_i32[:, None, :], (_VOCAB, 2, _QH)),
            tbit[:, :, None],
            jnp.broadcast_to(pad[:, None, 1:], (_VOCAB, 2, _RW - _QH - 1)),
        ],
        axis=2,
    ).reshape(2 * _VOCAB, _RW)

    gathered = _sc_gather(ids_eff, ext)
    g3 = gathered.reshape(_BATCH, _SEQ, _RW)

    posf = pos_table[:_SEQ].astype(jnp.float32) * pos_scale
    t0 = type_table[0].astype(jnp.float32) * type_scale
    t1 = type_table[1].astype(jnp.float32) * type_scale
    ptc = posf + t0[None, :]
    dt = (t1 - t0)[None, :]
    lnw = ln_weight[None, :]
    lnb = ln_bias[None, :]
    ws1 = word_scale.reshape(1)

    out = pl.pallas_call(
        _ln_body,
        grid=(_NB,),
        in_specs=[
            pl.BlockSpec(memory_space=pltpu.SMEM),
            pl.BlockSpec((_BB, _SEQ, _RW), lambda i: (i, 0, 0)),
            pl.BlockSpec((_SEQ, _HIDDEN), lambda i: (0, 0)),
            pl.BlockSpec((1, _HIDDEN), lambda i: (0, 0)),
            pl.BlockSpec((1, _HIDDEN), lambda i: (0, 0)),
            pl.BlockSpec((1, _HIDDEN), lambda i: (0, 0)),
        ],
        out_specs=pl.BlockSpec((_BB, _SEQ, _HIDDEN), lambda i: (i, 0, 0)),
        out_shape=jax.ShapeDtypeStruct((_BATCH, _SEQ, _HIDDEN), jnp.float32),
    )(ws1, g3, ptc, dt, lnw, lnb)

    return out


# speedup vs baseline: 5.3416x; 1.0931x over previous
import functools

import jax
import jax.numpy as jnp
from jax import lax
from jax.experimental import pallas as pl
from jax.experimental.pallas import tpu as pltpu
from jax.experimental.pallas import tpu_sc as plsc

_VOCAB = 30522
_HIDDEN = 768
_SEQ = 128
_BATCH = 1024
_EPS = 1e-12
_NR_ITERS = 8

_QH = _HIDDEN // 4
_RW = 256
_TOK = _BATCH * _SEQ
_NW = 32
_TPW = _TOK // _NW
_CH = 128
_NCH = _TPW // _CH

_BB = 8
_NB = _BATCH // _BB


def _sc_gather(ids3, table_i32):
    info = plsc.get_sparse_core_info()
    nc = info.num_cores

    mesh = plsc.VectorSubcoreMesh(core_axis_name="c", subcore_axis_name="s")

    @functools.partial(
        pl.kernel,
        mesh=mesh,
        out_type=jax.ShapeDtypeStruct((_TOK, _RW), jnp.int32),
        scratch_types=[
            pltpu.VMEM((_NCH, _CH), jnp.int32),
            pltpu.VMEM((2, _CH, _RW), jnp.int32),
            pltpu.SemaphoreType.DMA,
            pltpu.SemaphoreType.DMA,
        ],
    )
    def gk(ids_hbm, tab_hbm, out_hbm, idx_v, rows_v, gsem, ssem):
        wid = lax.axis_index("s") * nc + lax.axis_index("c")
        base = wid * _TPW
        pltpu.sync_copy(ids_hbm.at[wid], idx_v)

        pltpu.async_copy(tab_hbm.at[idx_v.at[0]], rows_v.at[0], gsem)

        def body(c, carry):
            buf = lax.rem(c, 2)

            @pl.when(c + 1 < _NCH)
            def _prefetch():
                pltpu.async_copy(
                    tab_hbm.at[idx_v.at[c + 1]], rows_v.at[1 - buf], gsem
                )

            pltpu.make_async_copy(
                tab_hbm.at[idx_v.at[c]], rows_v.at[buf], gsem
            ).wait()
            pltpu.async_copy(
                rows_v.at[buf], out_hbm.at[pl.ds(base + c * _CH, _CH)], ssem
            ).wait()
            return carry

        lax.fori_loop(0, _NCH, body, 0)

    return gk(ids3, table_i32)


def _ln_body(ws_ref, g_ref, ptc0_ref, dt_ref, lnw_ref, lnb_ref, out_ref,
             svar_ref, sr_ref):
    w32 = g_ref[...]
    ws = ws_ref[0]
    ttf = w32[:, :, _QH:_QH + 1].astype(jnp.float32)
    q = w32[:, :, :_QH]
    pieces = []
    ssum = None
    ssq = None
    for m in range(4):
        b = lax.shift_right_arithmetic(
            lax.shift_left(q, 24 - 8 * m), 24
        ).astype(jnp.float32)
        p0 = ptc0_ref[:, m * _QH:(m + 1) * _QH][None]
        dt = dt_ref[:, m * _QH:(m + 1) * _QH][None]
        em = (b + p0) + ttf * dt
        pieces.append(em)
    e = jnp.concatenate(pieces, axis=2)
    mean = jnp.sum(e, axis=2, keepdims=True) * (1.0 / _HIDDEN)
    var = jnp.sum(e * e, axis=2, keepdims=True) * (1.0 / _HIDDEN) - mean * mean
    svar_ref[...] = var.reshape(_BB, _SEQ)
    s = svar_ref[...] * (ws * ws)
    x = jnp.where(s > 1.0, s * 0.5, jnp.ones_like(s))
    for _ in range(_NR_ITERS):
        x = 0.5 * (x + s / (x + 1e-9))
    sr_ref[...] = 1.0 / (x + _EPS)
    r = sr_ref[...].reshape(_BB, _SEQ, 1)
    out_ref[...] = ((e - mean) * r) * lnw_ref[...][None] + lnb_ref[...][None]


def kernel(input_ids, token_type_ids, word_table, word_scale, pos_table,
           pos_scale, type_table, type_scale, ln_weight, ln_bias):
    ids_eff = (input_ids.astype(jnp.int32) * 2
               + token_type_ids.astype(jnp.int32)).reshape(_NW, _NCH, _CH)

    wt_t = word_table.reshape(_VOCAB, 4, _QH).transpose(0, 2, 1)
    wt_i32 = lax.bitcast_convert_type(wt_t, jnp.int32)
    pad = jnp.zeros((_VOCAB, _RW - _QH), jnp.int32)
    tbit = jnp.concatenate(
        [jnp.zeros((_VOCAB, 1), jnp.int32), jnp.ones((_VOCAB, 1), jnp.int32)],
        axis=1,
    )
    ext = jnp.concatenate(
        [
            jnp.broadcast_to(wt_i32[:, None, :], (_VOCAB, 2, _QH)),
            tbit[:, :, None],
            jnp.broadcast_to(pad[:, None, 1:], (_VOCAB, 2, _RW - _QH - 1)),
        ],
        axis=2,
    ).reshape(2 * _VOCAB, _RW)

    gathered = _sc_gather(ids_eff, ext)
    g3 = gathered.reshape(_BATCH, _SEQ, _RW)

    posf = pos_table[:_SEQ].astype(jnp.float32) * pos_scale
    t0 = type_table[0].astype(jnp.float32) * type_scale
    t1 = type_table[1].astype(jnp.float32) * type_scale
    ptc0 = (posf + t0[None, :]) / word_scale
    dt = ((t1 - t0) / word_scale)[None, :]
    lnw = (ln_weight * word_scale)[None, :]
    lnb = ln_bias[None, :]
    ws1 = word_scale.reshape(1)

    out = pl.pallas_call(
        _ln_body,
        grid=(_NB,),
        in_specs=[
            pl.BlockSpec(memory_space=pltpu.SMEM),
            pl.BlockSpec((_BB, _SEQ, _RW), lambda i: (i, 0, 0)),
            pl.BlockSpec((_SEQ, _HIDDEN), lambda i: (0, 0)),
            pl.BlockSpec((1, _HIDDEN), lambda i: (0, 0)),
            pl.BlockSpec((1, _HIDDEN), lambda i: (0, 0)),
            pl.BlockSpec((1, _HIDDEN), lambda i: (0, 0)),
        ],
        out_specs=pl.BlockSpec((_BB, _SEQ, _HIDDEN), lambda i: (i, 0, 0)),
        out_shape=jax.ShapeDtypeStruct((_BATCH, _SEQ, _HIDDEN), jnp.float32),
        scratch_shapes=[
            pltpu.VMEM((_BB, _SEQ), jnp.float32),
            pltpu.VMEM((_BB, _SEQ), jnp.float32),
        ],
    )(ws1, g3, ptc0, dt, lnw, lnb)

    return out


# speedup vs baseline: 9.3010x; 1.7413x over previous
import functools

import jax
import jax.numpy as jnp
from jax import lax
from jax.experimental import pallas as pl
from jax.experimental.pallas import tpu as pltpu
from jax.experimental.pallas import tpu_sc as plsc

_VOCAB = 30522
_HIDDEN = 768
_SEQ = 128
_BATCH = 1024
_EPS = 1e-12
_NR_ITERS = 8

_QH = _HIDDEN // 4
_RW = 256
_TOK = _BATCH * _SEQ
_NW = 32
_TPW = _TOK // _NW
_CH = 128
_NCH = _TPW // _CH

_BB = 8
_NB = _BATCH // _BB


def _sc_gather(ids3, table_i32):
    info = plsc.get_sparse_core_info()
    nc = info.num_cores

    mesh = plsc.VectorSubcoreMesh(core_axis_name="c", subcore_axis_name="s")

    @functools.partial(
        pl.kernel,
        mesh=mesh,
        out_type=jax.ShapeDtypeStruct((_TOK, _RW), jnp.int32),
        scratch_types=[
            pltpu.VMEM((_NCH, _CH), jnp.int32),
            pltpu.VMEM((2, _CH, _RW), jnp.int32),
            pltpu.SemaphoreType.DMA,
            pltpu.SemaphoreType.DMA,
        ],
    )
    def gk(ids_hbm, tab_hbm, out_hbm, idx_v, rows_v, gsem, ssem):
        wid = lax.axis_index("s") * nc + lax.axis_index("c")
        base = wid * _TPW
        pltpu.sync_copy(ids_hbm.at[wid], idx_v)

        pltpu.async_copy(tab_hbm.at[idx_v.at[0]], rows_v.at[0], gsem)

        def body(c, carry):
            buf = lax.rem(c, 2)

            @pl.when(c + 1 < _NCH)
            def _prefetch():
                pltpu.async_copy(
                    tab_hbm.at[idx_v.at[c + 1]], rows_v.at[1 - buf], gsem
                )

            pltpu.make_async_copy(
                tab_hbm.at[idx_v.at[c]], rows_v.at[buf], gsem
            ).wait()
            pltpu.async_copy(
                rows_v.at[buf], out_hbm.at[pl.ds(base + c * _CH, _CH)], ssem
            ).wait()
            return carry

        lax.fori_loop(0, _NCH, body, 0)

    return gk(ids3, table_i32)


def _ln_body(ws_ref, g_ref, ptc0_ref, dt_ref, lnw_ref, lnb_ref, out_ref,
             svar_ref, sr_ref):
    w32 = g_ref[...]
    ws = ws_ref[0]
    ttf = w32[:, :, _QH:_QH + 1].astype(jnp.float32)
    q = w32[:, :, :_QH]
    pieces = []
    ssum = None
    ssq = None
    for m in range(4):
        b = lax.shift_right_arithmetic(
            lax.shift_left(q, 24 - 8 * m), 24
        ).astype(jnp.float32)
        p0 = ptc0_ref[:, m * _QH:(m + 1) * _QH][None]
        dt = dt_ref[:, m * _QH:(m + 1) * _QH][None]
        em = (b + p0) + ttf * dt
        pieces.append(em)
    e = jnp.concatenate(pieces, axis=2)
    mean = jnp.sum(e, axis=2, keepdims=True) * (1.0 / _HIDDEN)
    var = jnp.sum(e * e, axis=2, keepdims=True) * (1.0 / _HIDDEN) - mean * mean
    svar_ref[...] = var.reshape(_BB, _SEQ)
    s = svar_ref[...] * (ws * ws)
    x = jnp.where(s > 1.0, s * 0.5, jnp.ones_like(s))
    for _ in range(_NR_ITERS):
        x = 0.5 * (x + s / (x + 1e-9))
    sr_ref[...] = 1.0 / (x + _EPS)
    r = sr_ref[...].reshape(_BB, _SEQ, 1)
    out_ref[...] = ((e - mean) * r) * lnw_ref[...][None] + lnb_ref[...][None]


_VB = 512
_NVB = 60
_VPAD = _VB * _NVB


def _build_body(w_ref, out_ref):
    y = pl.program_id(0)
    x = w_ref[...].astype(jnp.int32)
    p0 = x[:, 0 * _QH:1 * _QH] & 255
    p1 = x[:, 1 * _QH:2 * _QH] & 255
    p2 = x[:, 2 * _QH:3 * _QH] & 255
    p3 = x[:, 3 * _QH:4 * _QH]
    word = p0 | (p1 << 8) | (p2 << 16) | (p3 << 24)
    tcol = jnp.full((_VB, 1), y, jnp.int32)
    pad = jnp.zeros((_VB, _RW - _QH - 1), jnp.int32)
    out_ref[...] = jnp.concatenate([word, tcol, pad], axis=1)


def kernel(input_ids, token_type_ids, word_table, word_scale, pos_table,
           pos_scale, type_table, type_scale, ln_weight, ln_bias):
    ids_eff = (input_ids.astype(jnp.int32)
               + _VPAD * token_type_ids.astype(jnp.int32)
               ).reshape(_NW, _NCH, _CH)

    ext = pl.pallas_call(
        _build_body,
        grid=(2, _NVB),
        in_specs=[pl.BlockSpec((_VB, _HIDDEN), lambda y, i: (i, 0))],
        out_specs=pl.BlockSpec((_VB, _RW), lambda y, i: (y * _NVB + i, 0)),
        out_shape=jax.ShapeDtypeStruct((2 * _VPAD, _RW), jnp.int32),
    )(word_table)

    gathered = _sc_gather(ids_eff, ext)
    g3 = gathered.reshape(_BATCH, _SEQ, _RW)

    posf = pos_table[:_SEQ].astype(jnp.float32) * pos_scale
    t0 = type_table[0].astype(jnp.float32) * type_scale
    t1 = type_table[1].astype(jnp.float32) * type_scale
    ptc0 = (posf + t0[None, :]) / word_scale
    dt = ((t1 - t0) / word_scale)[None, :]
    lnw = (ln_weight * word_scale)[None, :]
    lnb = ln_bias[None, :]
    ws1 = word_scale.reshape(1)

    out = pl.pallas_call(
        _ln_body,
        grid=(_NB,),
        in_specs=[
            pl.BlockSpec(memory_space=pltpu.SMEM),
            pl.BlockSpec((_BB, _SEQ, _RW), lambda i: (i, 0, 0)),
            pl.BlockSpec((_SEQ, _HIDDEN), lambda i: (0, 0)),
            pl.BlockSpec((1, _HIDDEN), lambda i: (0, 0)),
            pl.BlockSpec((1, _HIDDEN), lambda i: (0, 0)),
            pl.BlockSpec((1, _HIDDEN), lambda i: (0, 0)),
        ],
        out_specs=pl.BlockSpec((_BB, _SEQ, _HIDDEN), lambda i: (i, 0, 0)),
        out_shape=jax.ShapeDtypeStruct((_BATCH, _SEQ, _HIDDEN), jnp.float32),
        scratch_shapes=[
            pltpu.VMEM((_BB, _SEQ), jnp.float32),
            pltpu.VMEM((_BB, _SEQ), jnp.float32),
        ],
    )(ws1, g3, ptc0, dt, lnw, lnb)

    return out


# speedup vs baseline: 12.1916x; 1.3108x over previous
import functools

import jax
import jax.numpy as jnp
from jax import lax
from jax.experimental import pallas as pl
from jax.experimental.pallas import tpu as pltpu
from jax.experimental.pallas import tpu_sc as plsc

_VOCAB = 30522
_HIDDEN = 768
_SEQ = 128
_BATCH = 1024
_EPS = 1e-12
_NR_ITERS = 8

_RW = 256
_PW = _HIDDEN // 3
_TOK = _BATCH * _SEQ
_NW = 32
_TPW = _TOK // _NW
_CH = 128
_NCH = _TPW // _CH

_BB = 8
_NB = _BATCH // _BB


def _sc_gather(ids3, table_i32):
    info = plsc.get_sparse_core_info()
    nc = info.num_cores

    mesh = plsc.VectorSubcoreMesh(core_axis_name="c", subcore_axis_name="s")

    @functools.partial(
        pl.kernel,
        mesh=mesh,
        out_type=jax.ShapeDtypeStruct((_TOK, _RW), jnp.int32),
        scratch_types=[
            pltpu.VMEM((_NCH, _CH), jnp.int32),
            pltpu.VMEM((2, _CH, _RW), jnp.int32),
            pltpu.SemaphoreType.DMA,
            pltpu.SemaphoreType.DMA,
        ],
    )
    def gk(ids_hbm, tab_hbm, out_hbm, idx_v, rows_v, gsem, ssem):
        wid = lax.axis_index("s") * nc + lax.axis_index("c")
        base = wid * _TPW
        pltpu.sync_copy(ids_hbm.at[wid], idx_v)

        pltpu.async_copy(tab_hbm.at[idx_v.at[0]], rows_v.at[0], gsem)

        def body(c, carry):
            buf = lax.rem(c, 2)

            @pl.when(c + 1 < _NCH)
            def _prefetch():
                pltpu.async_copy(
                    tab_hbm.at[idx_v.at[c + 1]], rows_v.at[1 - buf], gsem
                )

            pltpu.make_async_copy(
                tab_hbm.at[idx_v.at[c]], rows_v.at[buf], gsem
            ).wait()
            pltpu.async_copy(
                rows_v.at[buf], out_hbm.at[pl.ds(base + c * _CH, _CH)], ssem
            ).wait()
            return carry

        lax.fori_loop(0, _NCH, body, 0)

    return gk(ids3, table_i32)


def _ln_body(ws_ref, g_ref, tt_ref, ptc0_ref, dt_ref, lnw_ref, lnb_ref,
             out_ref, svar_ref, sr_ref):
    q = g_ref[...]
    ws = ws_ref[0]
    ttf = tt_ref[...].astype(jnp.float32)[:, :, None]
    pieces = []
    for m in range(3):
        b = lax.shift_right_arithmetic(
            lax.shift_left(q, 24 - 8 * m), 24
        ).astype(jnp.float32)
        p0 = ptc0_ref[:, m * _PW:(m + 1) * _PW][None]
        dt = dt_ref[:, m * _PW:(m + 1) * _PW][None]
        em = (b + p0) + ttf * dt
        pieces.append(em)
    e = jnp.concatenate(pieces, axis=2)
    mean = jnp.sum(e, axis=2, keepdims=True) * (1.0 / _HIDDEN)
    var = jnp.sum(e * e, axis=2, keepdims=True) * (1.0 / _HIDDEN) - mean * mean
    svar_ref[...] = var.reshape(_BB, _SEQ)
    s = svar_ref[...] * (ws * ws)
    x = jnp.where(s > 1.0, s * 0.5, jnp.ones_like(s))
    for _ in range(_NR_ITERS):
        x = 0.5 * (x + s / (x + 1e-9))
    sr_ref[...] = 1.0 / (x + _EPS)
    r = sr_ref[...].reshape(_BB, _SEQ, 1)
    out_ref[...] = ((e - mean) * r) * lnw_ref[...][None] + lnb_ref[...][None]


_VB = 512
_NVB = 60
_VPAD = _VB * _NVB


def _build_body(w_ref, out_ref):
    x = w_ref[...].astype(jnp.int32)
    p0 = x[:, 0 * _PW:1 * _PW] & 255
    p1 = x[:, 1 * _PW:2 * _PW] & 255
    p2 = x[:, 2 * _PW:3 * _PW] & 255
    out_ref[...] = p0 | (p1 << 8) | (p2 << 16)


def kernel(input_ids, token_type_ids, word_table, word_scale, pos_table,
           pos_scale, type_table, type_scale, ln_weight, ln_bias):
    ids_eff = input_ids.astype(jnp.int32).reshape(_NW, _NCH, _CH)

    ext = pl.pallas_call(
        _build_body,
        grid=(_NVB,),
        in_specs=[pl.BlockSpec((_VB, _HIDDEN), lambda i: (i, 0))],
        out_specs=pl.BlockSpec((_VB, _RW), lambda i: (i, 0)),
        out_shape=jax.ShapeDtypeStruct((_VPAD, _RW), jnp.int32),
    )(word_table)

    gathered = _sc_gather(ids_eff, ext)
    g3 = gathered.reshape(_BATCH, _SEQ, _RW)
    tt = token_type_ids.astype(jnp.int32)

    posf = pos_table[:_SEQ].astype(jnp.float32) * pos_scale
    t0 = type_table[0].astype(jnp.float32) * type_scale
    t1 = type_table[1].astype(jnp.float32) * type_scale
    ptc0 = (posf + t0[None, :]) / word_scale
    dt = ((t1 - t0) / word_scale)[None, :]
    lnw = (ln_weight * word_scale)[None, :]
    lnb = ln_bias[None, :]
    ws1 = word_scale.reshape(1)

    out = pl.pallas_call(
        _ln_body,
        grid=(_NB,),
        in_specs=[
            pl.BlockSpec(memory_space=pltpu.SMEM),
            pl.BlockSpec((_BB, _SEQ, _RW), lambda i: (i, 0, 0)),
            pl.BlockSpec((_BB, _SEQ), lambda i: (i, 0)),
            pl.BlockSpec((_SEQ, _HIDDEN), lambda i: (0, 0)),
            pl.BlockSpec((1, _HIDDEN), lambda i: (0, 0)),
            pl.BlockSpec((1, _HIDDEN), lambda i: (0, 0)),
            pl.BlockSpec((1, _HIDDEN), lambda i: (0, 0)),
        ],
        out_specs=pl.BlockSpec((_BB, _SEQ, _HIDDEN), lambda i: (i, 0, 0)),
        out_shape=jax.ShapeDtypeStruct((_BATCH, _SEQ, _HIDDEN), jnp.float32),
        scratch_shapes=[
            pltpu.VMEM((_BB, _SEQ), jnp.float32),
            pltpu.VMEM((_BB, _SEQ), jnp.float32),
        ],
    )(ws1, g3, tt, ptc0, dt, lnw, lnb)

    return out


# speedup vs baseline: 14.0932x; 1.1560x over previous
import functools

import jax
import jax.numpy as jnp
from jax import lax
from jax.experimental import pallas as pl
from jax.experimental.pallas import tpu as pltpu
from jax.experimental.pallas import tpu_sc as plsc

_VOCAB = 30522
_HIDDEN = 768
_SEQ = 128
_BATCH = 1024
_EPS = 1e-12
_NR_ITERS = 8

_RW = 256
_PW = _HIDDEN // 3
_TOK = _BATCH * _SEQ
_NW = 32
_TPW = _TOK // _NW
_CH = 128
_NCH = _TPW // _CH

_BB = 32
_NB = _BATCH // _BB


def _sc_gather(ids3, table_i32):
    info = plsc.get_sparse_core_info()
    nc = info.num_cores

    mesh = plsc.VectorSubcoreMesh(core_axis_name="c", subcore_axis_name="s")

    @functools.partial(
        pl.kernel,
        mesh=mesh,
        out_type=jax.ShapeDtypeStruct((_TOK, _RW), jnp.int32),
        scratch_types=[
            pltpu.VMEM((_NCH, _CH), jnp.int32),
            pltpu.VMEM((2, _CH, _RW), jnp.int32),
            pltpu.SemaphoreType.DMA,
            pltpu.SemaphoreType.DMA,
        ],
    )
    def gk(ids_hbm, tab_hbm, out_hbm, idx_v, rows_v, gsem, ssem):
        wid = lax.axis_index("s") * nc + lax.axis_index("c")
        base = wid * _TPW
        pltpu.sync_copy(ids_hbm.at[wid], idx_v)

        pltpu.async_copy(tab_hbm.at[idx_v.at[0]], rows_v.at[0], gsem)

        def body(c, carry):
            buf = lax.rem(c, 2)

            @pl.when(c + 1 < _NCH)
            def _prefetch():
                pltpu.async_copy(
                    tab_hbm.at[idx_v.at[c + 1]], rows_v.at[1 - buf], gsem
                )

            pltpu.make_async_copy(
                tab_hbm.at[idx_v.at[c]], rows_v.at[buf], gsem
            ).wait()
            pltpu.async_copy(
                rows_v.at[buf], out_hbm.at[pl.ds(base + c * _CH, _CH)], ssem
            ).wait()
            return carry

        lax.fori_loop(0, _NCH, body, 0)

    return gk(ids3, table_i32)


def _ln_body(ws_ref, g_ref, tt_ref, ptc0_ref, dt_ref, lnw_ref, lnb_ref,
             out_ref, svar_ref, sr_ref):
    q = g_ref[...]
    ws = ws_ref[0]
    ttf = tt_ref[...].astype(jnp.float32)[:, :, None]
    pieces = []
    for m in range(3):
        b = lax.shift_right_arithmetic(
            lax.shift_left(q, 24 - 8 * m), 24
        ).astype(jnp.float32)
        p0 = ptc0_ref[:, m * _PW:(m + 1) * _PW][None]
        dt = dt_ref[:, m * _PW:(m + 1) * _PW][None]
        em = (b + p0) + ttf * dt
        pieces.append(em)
    e = jnp.concatenate(pieces, axis=2)
    mean = jnp.sum(e, axis=2, keepdims=True) * (1.0 / _HIDDEN)
    var = jnp.sum(e * e, axis=2, keepdims=True) * (1.0 / _HIDDEN) - mean * mean
    svar_ref[...] = var.reshape(_BB, _SEQ)
    s = svar_ref[...] * (ws * ws)
    x = jnp.where(s > 1.0, s * 0.5, jnp.ones_like(s))
    for _ in range(_NR_ITERS):
        x = 0.5 * (x + s / (x + 1e-9))
    sr_ref[...] = 1.0 / (x + _EPS)
    r = sr_ref[...].reshape(_BB, _SEQ, 1)
    out_ref[...] = ((e - mean) * r) * lnw_ref[...][None] + lnb_ref[...][None]


_VB = 512
_NVB = 60
_VPAD = _VB * _NVB


def _build_body(w_ref, out_ref):
    x = w_ref[...].astype(jnp.int32)
    p0 = x[:, 0 * _PW:1 * _PW] & 255
    p1 = x[:, 1 * _PW:2 * _PW] & 255
    p2 = x[:, 2 * _PW:3 * _PW] & 255
    out_ref[...] = p0 | (p1 << 8) | (p2 << 16)


def kernel(input_ids, token_type_ids, word_table, word_scale, pos_table,
           pos_scale, type_table, type_scale, ln_weight, ln_bias):
    ids_eff = input_ids.astype(jnp.int32).reshape(_NW, _NCH, _CH)

    ext = pl.pallas_call(
        _build_body,
        grid=(_NVB,),
        in_specs=[pl.BlockSpec((_VB, _HIDDEN), lambda i: (i, 0))],
        out_specs=pl.BlockSpec((_VB, _RW), lambda i: (i, 0)),
        out_shape=jax.ShapeDtypeStruct((_VPAD, _RW), jnp.int32),
    )(word_table)

    gathered = _sc_gather(ids_eff, ext)
    g3 = gathered.reshape(_BATCH, _SEQ, _RW)
    tt = token_type_ids.astype(jnp.int32)

    posf = pos_table[:_SEQ].astype(jnp.float32) * pos_scale
    t0 = type_table[0].astype(jnp.float32) * type_scale
    t1 = type_table[1].astype(jnp.float32) * type_scale
    ptc0 = (posf + t0[None, :]) / word_scale
    dt = ((t1 - t0) / word_scale)[None, :]
    lnw = (ln_weight * word_scale)[None, :]
    lnb = ln_bias[None, :]
    ws1 = word_scale.reshape(1)

    out = pl.pallas_call(
        _ln_body,
        grid=(_NB,),
        in_specs=[
            pl.BlockSpec(memory_space=pltpu.SMEM),
            pl.BlockSpec((_BB, _SEQ, _RW), lambda i: (i, 0, 0)),
            pl.BlockSpec((_BB, _SEQ), lambda i: (i, 0)),
            pl.BlockSpec((_SEQ, _HIDDEN), lambda i: (0, 0)),
            pl.BlockSpec((1, _HIDDEN), lambda i: (0, 0)),
            pl.BlockSpec((1, _HIDDEN), lambda i: (0, 0)),
            pl.BlockSpec((1, _HIDDEN), lambda i: (0, 0)),
        ],
        out_specs=pl.BlockSpec((_BB, _SEQ, _HIDDEN), lambda i: (i, 0, 0)),
        out_shape=jax.ShapeDtypeStruct((_BATCH, _SEQ, _HIDDEN), jnp.float32),
        scratch_shapes=[
            pltpu.VMEM((_BB, _SEQ), jnp.float32),
            pltpu.VMEM((_BB, _SEQ), jnp.float32),
        ],
    )(ws1, g3, tt, ptc0, dt, lnw, lnb)

    return out
